# baseline (speedup 1.0000x reference)
import functools

import jax
import jax.numpy as jnp
from jax import lax
from jax.experimental import pallas as pl
from jax.experimental.pallas import tpu as pltpu
from jax.experimental.pallas import tpu_sc as plsc

B = 16384
Z_DIM = 64
ZP = 128
NC = 2
NS = 16
NW = NC * NS
BPW = B // NW
CHUNK = 128
NCHUNK = BPW // CHUNK
L = 16


@functools.partial(
    pl.kernel,
    mesh=plsc.VectorSubcoreMesh(core_axis_name="c", subcore_axis_name="s"),
    out_type=(
        jax.ShapeDtypeStruct((B, ZP), jnp.float32),
        jax.ShapeDtypeStruct((B,), jnp.float32),
        jax.ShapeDtypeStruct((B,), jnp.float32),
    ),
    scratch_types=[
        pltpu.VMEM((BPW,), jnp.int32),
        pltpu.VMEM((BPW,), jnp.int32),
        pltpu.VMEM((BPW, ZP), jnp.float32),
        pltpu.VMEM((BPW,), jnp.float32),
        pltpu.VMEM((BPW,), jnp.float32),
        pltpu.SemaphoreType.DMA,
    ],
    compiler_params=pltpu.CompilerParams(use_tc_tiling_on_sc=True),
)
def _sc_gather(idx_hbm, emb2_hbm, inh_hbm, det_hbm,
               z_out, inh_out, det_out,
               idx_v, row_v, z2_v, inh_v, det_v, sem):
    wid = lax.axis_index("s") * NC + lax.axis_index("c")
    base = wid * BPW
    pltpu.sync_copy(idx_hbm.at[pl.ds(base, BPW)], idx_v)
    for k in range(BPW // L):
        sl = pl.ds(k * L, L)
        row_v[sl] = lax.shift_right_logical(idx_v[sl], 1)
    copies = []
    for c in range(NCHUNK):
        sl = pl.ds(c * CHUNK, CHUNK)
        copies.append(pltpu.async_copy(emb2_hbm.at[row_v.at[sl]], z2_v.at[sl], sem))
        copies.append(pltpu.async_copy(inh_hbm.at[idx_v.at[sl]], inh_v.at[sl], sem))
        copies.append(pltpu.async_copy(det_hbm.at[idx_v.at[sl]], det_v.at[sl], sem))
    for cp in copies:
        cp.wait()

    def body(g, carry):
        offs = (idx_v[pl.ds(g * L, L)] & 1) * Z_DIM
        for t in range(L):
            j = g * L + t
            off = offs[t]
            for k in range(Z_DIM // L):
                z2_v[j, pl.ds(k * L, L)] = z2_v[j, pl.ds(off + k * L, L)]
        return carry

    lax.fori_loop(0, BPW // L, body, 0)
    pltpu.sync_copy(z2_v, z_out.at[pl.ds(base, BPW)])
    pltpu.sync_copy(inh_v, inh_out.at[pl.ds(base, BPW)])
    pltpu.sync_copy(det_v, det_out.at[pl.ds(base, BPW)])


def kernel(piano_model, emb_w, inharm_w, detune_w):
    idx = piano_model.astype(jnp.int32)
    emb2 = emb_w.reshape(emb_w.shape[0] // 2, ZP)
    z128, inh, det = _sc_gather(idx, emb2,
                                inharm_w.reshape(-1), detune_w.reshape(-1))
    return (z128[:, None, :Z_DIM],
            inh.reshape(B, 1, 1),
            det.reshape(B, 1, 1))
